# baseline (speedup 1.0000x reference)
import functools

import jax
import jax.numpy as jnp
from jax import lax
from jax.experimental import pallas as pl
from jax.experimental.pallas import tpu as pltpu
from jax.experimental.pallas import tpu_sc as plsc

N_TOKENS = 16384
D_MODEL = 2048
N_EXP = 64
N_TOPICS = 4
BN = 2048


def _tc_body(u_ref, wg_ref, wh_ref, bh_ref,
             head_ref, s_ref, aux_ref,
             dens_ref, prox_ref):
    step = pl.program_id(0)
    nsteps = pl.num_programs(0)

    @pl.when(step == 0)
    def _init():
        dens_ref[...] = jnp.zeros_like(dens_ref)
        prox_ref[...] = jnp.zeros_like(prox_ref)

    logits = jnp.dot(u_ref[...], wg_ref[...],
                     preferred_element_type=jnp.float32)
    m = jnp.max(logits, axis=-1, keepdims=True)
    ex = jnp.exp(logits - m)
    s = ex / jnp.sum(ex, axis=-1, keepdims=True)
    s_ref[...] = s

    head_ref[...] = (jnp.dot(s, wh_ref[...],
                             preferred_element_type=jnp.float32)
                     + bh_ref[...])

    iota = lax.broadcasted_iota(jnp.int32, s.shape, 1)
    m1 = jnp.max(s, axis=-1, keepdims=True)
    i1 = jnp.min(jnp.where(s == m1, iota, N_EXP), axis=-1, keepdims=True)
    s2 = jnp.where(iota == i1, -jnp.inf, s)
    m2 = jnp.max(s2, axis=-1, keepdims=True)
    i2 = jnp.min(jnp.where(s2 == m2, iota, N_EXP), axis=-1, keepdims=True)
    hit = ((iota == i1) | (iota == i2)).astype(jnp.float32)
    dens_ref[...] += jnp.sum(hit, axis=0, keepdims=True)
    prox_ref[...] += jnp.sum(s, axis=0, keepdims=True)

    @pl.when(step == nsteps - 1)
    def _finish():
        n = jnp.float32(N_TOKENS)
        aux_ref[...] = (jnp.float32(N_EXP)
                        * jnp.sum(dens_ref[...] * prox_ref[...],
                                  axis=1, keepdims=True) / (n * n))


@jax.jit
def _tc_dense(u, W_g, W_h, b_h2):
    grid = (N_TOKENS // BN,)
    return pl.pallas_call(
        _tc_body,
        grid=grid,
        in_specs=[
            pl.BlockSpec((BN, D_MODEL), lambda i: (i, 0)),
            pl.BlockSpec((D_MODEL, N_EXP), lambda i: (0, 0)),
            pl.BlockSpec((N_EXP, N_TOPICS), lambda i: (0, 0)),
            pl.BlockSpec((1, N_TOPICS), lambda i: (0, 0)),
        ],
        out_specs=[
            pl.BlockSpec((BN, N_TOPICS), lambda i: (i, 0)),
            pl.BlockSpec((BN, N_EXP), lambda i: (i, 0)),
            pl.BlockSpec((1, 1), lambda i: (0, 0)),
        ],
        out_shape=[
            jax.ShapeDtypeStruct((N_TOKENS, N_TOPICS), jnp.float32),
            jax.ShapeDtypeStruct((N_TOKENS, N_EXP), jnp.float32),
            jax.ShapeDtypeStruct((1, 1), jnp.float32),
        ],
        scratch_shapes=[
            pltpu.VMEM((1, N_EXP), jnp.float32),
            pltpu.VMEM((1, N_EXP), jnp.float32),
        ],
    )(u, W_g, W_h, b_h2)


_NC = 2
_NS = 16
_L = 16
_NW = _NC * _NS
_TOK_W = N_TOKENS // _NW
_GROUPS = _TOK_W // _L


def _sc_topk_body(s_hbm, idx_hbm, s_v, idx_v):
    wid = lax.axis_index("s") * _NC + lax.axis_index("c")
    pltpu.sync_copy(s_hbm.at[pl.ds(wid * _TOK_W * N_EXP, _TOK_W * N_EXP)],
                    s_v)

    lanes = lax.iota(jnp.int32, _L)

    def group(g, carry):
        srow = (g * _L + lanes) * N_EXP
        m1 = jnp.full((_L,), -jnp.inf, jnp.float32)
        m2 = jnp.full((_L,), -jnp.inf, jnp.float32)
        i1 = jnp.zeros((_L,), jnp.int32)
        i2 = jnp.zeros((_L,), jnp.int32)
        for e in range(N_EXP):
            e_vec = jnp.full((_L,), e, jnp.int32)
            v = plsc.load_gather(s_v, [srow + e_vec])
            gt1 = v > m1
            gt2 = v > m2
            m2 = jnp.where(gt1, m1, jnp.where(gt2, v, m2))
            i2 = jnp.where(gt1, i1, jnp.where(gt2, e_vec, i2))
            m1 = jnp.where(gt1, v, m1)
            i1 = jnp.where(gt1, e_vec, i1)
        pos = (g * _L + lanes) * 2
        plsc.store_scatter(idx_v, [pos], i1)
        plsc.store_scatter(idx_v, [pos + 1], i2)
        return carry

    lax.fori_loop(0, _GROUPS, group, 0)
    pltpu.sync_copy(idx_v, idx_hbm.at[pl.ds(wid * _TOK_W * 2, _TOK_W * 2)])


@jax.jit
def _sc_topk(s_flat):
    mesh = plsc.VectorSubcoreMesh(core_axis_name="c", subcore_axis_name="s")
    return pl.kernel(
        _sc_topk_body,
        mesh=mesh,
        out_type=jax.ShapeDtypeStruct((N_TOKENS * 2,), jnp.int32),
        scratch_types=[
            pltpu.VMEM((_TOK_W * N_EXP,), jnp.float32),
            pltpu.VMEM((_TOK_W * 2,), jnp.int32),
        ],
        compiler_params=pltpu.CompilerParams(needs_layout_passes=False),
    )(s_flat)


def kernel(u, W_g, W_h, b_h):
    head, s, aux = _tc_dense(u, W_g, W_h, b_h.reshape(1, N_TOPICS))
    idx = _sc_topk(s.reshape(-1)).reshape(N_TOKENS, 2)
    return (head, aux.reshape(()), idx)


# speedup vs baseline: 1.1722x; 1.1722x over previous
import functools

import jax
import jax.numpy as jnp
from jax import lax
from jax.experimental import pallas as pl
from jax.experimental.pallas import tpu as pltpu
from jax.experimental.pallas import tpu_sc as plsc

N_TOKENS = 16384
D_MODEL = 2048
N_EXP = 64
N_TOPICS = 4
BN = 2048


def _tc_body(u_ref, wg_ref, wh_ref, bh_ref,
             head_ref, st_ref, aux_ref,
             dens_ref, prox_ref):
    step = pl.program_id(0)
    nsteps = pl.num_programs(0)

    @pl.when(step == 0)
    def _init():
        dens_ref[...] = jnp.zeros_like(dens_ref)
        prox_ref[...] = jnp.zeros_like(prox_ref)

    logits = jnp.dot(u_ref[...], wg_ref[...],
                     preferred_element_type=jnp.float32)
    m = jnp.max(logits, axis=-1, keepdims=True)
    ex = jnp.exp(logits - m)
    s = ex / jnp.sum(ex, axis=-1, keepdims=True)
    st_ref[...] = s.T

    head_ref[...] = (jnp.dot(s, wh_ref[...],
                             preferred_element_type=jnp.float32)
                     + bh_ref[...])

    iota = lax.broadcasted_iota(jnp.int32, s.shape, 1)
    m1 = jnp.max(s, axis=-1, keepdims=True)
    i1 = jnp.min(jnp.where(s == m1, iota, N_EXP), axis=-1, keepdims=True)
    s2 = jnp.where(iota == i1, -jnp.inf, s)
    m2 = jnp.max(s2, axis=-1, keepdims=True)
    i2 = jnp.min(jnp.where(s2 == m2, iota, N_EXP), axis=-1, keepdims=True)
    hit = ((iota == i1) | (iota == i2)).astype(jnp.float32)
    dens_ref[...] += jnp.sum(hit, axis=0, keepdims=True)
    prox_ref[...] += jnp.sum(s, axis=0, keepdims=True)

    @pl.when(step == nsteps - 1)
    def _finish():
        n = jnp.float32(N_TOKENS)
        aux_ref[...] = (jnp.float32(N_EXP)
                        * jnp.sum(dens_ref[...] * prox_ref[...],
                                  axis=1, keepdims=True) / (n * n))


@jax.jit
def _tc_dense(u, W_g, W_h, b_h2):
    grid = (N_TOKENS // BN,)
    return pl.pallas_call(
        _tc_body,
        grid=grid,
        in_specs=[
            pl.BlockSpec((BN, D_MODEL), lambda i: (i, 0)),
            pl.BlockSpec((D_MODEL, N_EXP), lambda i: (0, 0)),
            pl.BlockSpec((N_EXP, N_TOPICS), lambda i: (0, 0)),
            pl.BlockSpec((1, N_TOPICS), lambda i: (0, 0)),
        ],
        out_specs=[
            pl.BlockSpec((BN, N_TOPICS), lambda i: (i, 0)),
            pl.BlockSpec((N_EXP, BN), lambda i: (0, i)),
            pl.BlockSpec((1, 1), lambda i: (0, 0)),
        ],
        out_shape=[
            jax.ShapeDtypeStruct((N_TOKENS, N_TOPICS), jnp.float32),
            jax.ShapeDtypeStruct((N_EXP, N_TOKENS), jnp.float32),
            jax.ShapeDtypeStruct((1, 1), jnp.float32),
        ],
        scratch_shapes=[
            pltpu.VMEM((1, N_EXP), jnp.float32),
            pltpu.VMEM((1, N_EXP), jnp.float32),
        ],
    )(u, W_g, W_h, b_h2)


_NC = 2
_NS = 16
_L = 16
_NW = _NC * _NS
_TOK_W = N_TOKENS // _NW
_GROUPS = _TOK_W // _L
_STRIPES = 4
_SE = N_EXP // _STRIPES


def _merge_top2(lo, hi):
    am1, ai1, am2, ai2 = lo
    bm1, bi1, bm2, bi2 = hi
    a_first = am1 >= bm1
    m1 = jnp.where(a_first, am1, bm1)
    i1 = jnp.where(a_first, ai1, bi1)
    a2_next = am2 >= bm1
    b2_next = am1 >= bm2
    m2 = jnp.where(a_first,
                   jnp.where(a2_next, am2, bm1),
                   jnp.where(b2_next, am1, bm2))
    i2 = jnp.where(a_first,
                   jnp.where(a2_next, ai2, bi1),
                   jnp.where(b2_next, ai1, bi2))
    return m1, i1, m2, i2


def _sc_topk_body(st_hbm, idx_hbm, s_v, idx_v):
    wid = lax.axis_index("s") * _NC + lax.axis_index("c")
    base = wid * _TOK_W
    pltpu.sync_copy(st_hbm.at[:, pl.ds(base, _TOK_W)], s_v)

    lanes = lax.iota(jnp.int32, _L)

    def group(g, carry):
        off = g * _L
        tops = []
        for k in range(_STRIPES):
            e0 = k * _SE
            m1 = s_v[e0, pl.ds(off, _L)]
            i1 = jnp.full((_L,), e0, jnp.int32)
            m2 = jnp.full((_L,), -jnp.inf, jnp.float32)
            i2 = jnp.full((_L,), e0, jnp.int32)
            for e in range(e0 + 1, e0 + _SE):
                v = s_v[e, pl.ds(off, _L)]
                e_vec = jnp.full((_L,), e, jnp.int32)
                gt1 = v > m1
                gt2 = v > m2
                m2 = jnp.where(gt1, m1, jnp.where(gt2, v, m2))
                i2 = jnp.where(gt1, i1, jnp.where(gt2, e_vec, i2))
                m1 = jnp.where(gt1, v, m1)
                i1 = jnp.where(gt1, e_vec, i1)
            tops.append((m1, i1, m2, i2))
        t01 = _merge_top2(tops[0], tops[1])
        t23 = _merge_top2(tops[2], tops[3])
        _, i1, _, i2 = _merge_top2(t01, t23)
        pos = (g * _L + lanes) * 2
        plsc.store_scatter(idx_v, [pos], i1)
        plsc.store_scatter(idx_v, [pos + 1], i2)
        return carry

    lax.fori_loop(0, _GROUPS, group, 0)
    pltpu.sync_copy(idx_v, idx_hbm.at[pl.ds(base * 2, _TOK_W * 2)])


@jax.jit
def _sc_topk(s_t):
    mesh = plsc.VectorSubcoreMesh(core_axis_name="c", subcore_axis_name="s")
    return pl.kernel(
        _sc_topk_body,
        mesh=mesh,
        out_type=jax.ShapeDtypeStruct((N_TOKENS * 2,), jnp.int32),
        scratch_types=[
            pltpu.VMEM((N_EXP, _TOK_W), jnp.float32),
            pltpu.VMEM((_TOK_W * 2,), jnp.int32),
        ],
        compiler_params=pltpu.CompilerParams(needs_layout_passes=False),
    )(s_t)


def kernel(u, W_g, W_h, b_h):
    head, s_t, aux = _tc_dense(u, W_g, W_h, b_h.reshape(1, N_TOPICS))
    idx = _sc_topk(s_t).reshape(N_TOKENS, 2)
    return (head, aux.reshape(()), idx)
